# baseline (speedup 1.0000x reference)
import functools

import jax
import jax.numpy as jnp
from jax import lax
from jax.experimental import pallas as pl
from jax.experimental.pallas import tpu as pltpu
from jax.experimental.pallas import tpu_sc as plsc

N = 10000
E = 320000
D = 128
G = 128

NC = 2
NS = 16
NW = NC * NS

CH = 128
NH = 2
HC = 40
NCHUNK = NH * HC
EPAD = NW * NCHUNK * CH
NPAD = 10240
RPW = NPAD // NS
ZB = RPW // 2
OPW = NPAD // NS


def _sc_agg_body(z_hbm, src_hbm, dst_hbm, out_hbm,
                 src_v, dst_v, rows_v, rows_w, agg_sh,
                 sem, sem2, ssem, ssem2):
    cid = lax.axis_index("c")
    sid = lax.axis_index("s")

    zeros16 = jnp.zeros((16,), jnp.float32)

    def zero_body(i, carry):
        for j in range(D // 16):
            rows_v[i, pl.ds(j * 16, 16)] = zeros16
        return carry

    lax.fori_loop(0, CH, zero_body, 0)
    for k in range(RPW // CH):
        pltpu.sync_copy(rows_v, agg_sh.at[pl.ds(sid * RPW + k * CH, CH)])

    plsc.subcore_barrier()

    def drain(buf, s):
        pltpu.make_async_copy(z_hbm.at[pl.ds(0, CH)], buf, s).wait()

    for h in range(NH):
        pltpu.sync_copy(src_hbm.at[cid, sid, h], src_v)
        pltpu.sync_copy(dst_hbm.at[cid, sid, h], dst_v)
        pltpu.async_copy(z_hbm.at[src_v.at[0]], rows_v, sem)
        pltpu.async_copy(z_hbm.at[src_v.at[1]], rows_w, sem2)

        def edge_body(k, carry):
            j = 2 * k
            drain(rows_v, sem)
            pltpu.async_copy(rows_v, agg_sh.at[dst_v.at[j]], ssem, add=True)
            drain(rows_w, sem2)
            pltpu.async_copy(rows_w, agg_sh.at[dst_v.at[j + 1]], ssem2,
                             add=True)
            drain(rows_v, ssem)
            pltpu.async_copy(z_hbm.at[src_v.at[(j + 2) % HC]], rows_v, sem)
            drain(rows_w, ssem2)
            pltpu.async_copy(z_hbm.at[src_v.at[(j + 3) % HC]], rows_w, sem2)
            return carry

        lax.fori_loop(0, HC // 2, edge_body, 0)
        drain(rows_v, sem)
        drain(rows_w, sem2)

    plsc.subcore_barrier()

    pltpu.sync_copy(agg_sh.at[pl.ds(sid * OPW, OPW)],
                    out_hbm.at[cid, pl.ds(sid * OPW, OPW)])


@functools.cache
def _get_sc_agg():
    mesh = plsc.VectorSubcoreMesh(core_axis_name="c", subcore_axis_name="s",
                                  num_cores=NC, num_subcores=NS)
    return pl.kernel(
        _sc_agg_body,
        out_type=jax.ShapeDtypeStruct((NC, NPAD, D), jnp.float32),
        mesh=mesh,
        scratch_types=[
            pltpu.VMEM((HC, CH), jnp.int32),
            pltpu.VMEM((HC, CH), jnp.int32),
            pltpu.VMEM((CH, D), jnp.float32),
            pltpu.VMEM((CH, D), jnp.float32),
            pltpu.VMEM_SHARED((NPAD, D), jnp.float32),
            pltpu.SemaphoreType.DMA,
            pltpu.SemaphoreType.DMA,
            pltpu.SemaphoreType.DMA,
            pltpu.SemaphoreType.DMA,
        ],
    )


def _tc_layer_body(z_ref, a0_ref, a1_ref, w1_ref, b1_ref, w2_ref, b2_ref,
                   gm_ref, bt_ref, out_ref):
    h = z_ref[...] + a0_ref[:N, :] + a1_ref[:N, :]
    h = jnp.maximum(jnp.dot(h, w1_ref[...],
                            preferred_element_type=jnp.float32) + b1_ref[...], 0.0)
    h = jnp.dot(h, w2_ref[...], preferred_element_type=jnp.float32) + b2_ref[...]
    h = jnp.maximum(h, 0.0)
    mean = jnp.mean(h, axis=0, keepdims=True)
    var = jnp.mean((h - mean) ** 2, axis=0, keepdims=True)
    out_ref[...] = (h - mean) * lax.rsqrt(var + 1e-5) * gm_ref[...] + bt_ref[...]


def _tc_layer(z, a0, a1, w1, b1, w2, b2, gm, bt):
    return pl.pallas_call(
        _tc_layer_body,
        out_shape=jax.ShapeDtypeStruct((N, D), jnp.float32),
    )(z, a0, a1, w1, b1, w2, b2, gm, bt)


def _tc_final_body(z1_ref, a0_ref, a1_ref, w1_ref, b1_ref, w2_ref, b2_ref,
                   gm_ref, bt_ref, batch_ref, z2_ref, g1_ref, g2_ref):
    h = z1_ref[...] + a0_ref[:N, :] + a1_ref[:N, :]
    h = jnp.maximum(jnp.dot(h, w1_ref[...],
                            preferred_element_type=jnp.float32) + b1_ref[...], 0.0)
    h = jnp.dot(h, w2_ref[...], preferred_element_type=jnp.float32) + b2_ref[...]
    h = jnp.maximum(h, 0.0)
    mean = jnp.mean(h, axis=0, keepdims=True)
    var = jnp.mean((h - mean) ** 2, axis=0, keepdims=True)
    z2 = (h - mean) * lax.rsqrt(var + 1e-5) * gm_ref[...] + bt_ref[...]
    z2_ref[...] = z2
    gid = lax.broadcasted_iota(jnp.int32, (G, 1), 0)
    onehot = (batch_ref[...] == gid).astype(jnp.float32)
    g1_ref[...] = jnp.dot(onehot, z1_ref[...], preferred_element_type=jnp.float32)
    g2_ref[...] = jnp.dot(onehot, z2, preferred_element_type=jnp.float32)


def _tc_final(z1, a0, a1, w1, b1, w2, b2, gm, bt, batch2d):
    return pl.pallas_call(
        _tc_final_body,
        out_shape=(
            jax.ShapeDtypeStruct((N, D), jnp.float32),
            jax.ShapeDtypeStruct((G, D), jnp.float32),
            jax.ShapeDtypeStruct((G, D), jnp.float32),
        ),
    )(z1, a0, a1, w1, b1, w2, b2, gm, bt, batch2d)


@jax.jit
def kernel(x, edge_index, batch, W1_0, b1_0, W2_0, b2_0, gamma_0, beta_0,
           W1_1, b1_1, W2_1, b2_1, gamma_1, beta_1):
    pad = EPAD - E
    src = jnp.concatenate([edge_index[0], jnp.zeros((pad,), jnp.int32)])
    dst = jnp.concatenate(
        [edge_index[1],
         N + (jnp.arange(pad, dtype=jnp.int32) % (NPAD - N))])
    src = src.reshape(NC, NS, NH, HC, CH)[::-1]
    dst = dst.reshape(NC, NS, NH, HC, CH)[::-1]

    b1_0r, b2_0r = b1_0.reshape(1, D), b2_0.reshape(1, D)
    g0r, be0r = gamma_0.reshape(1, D), beta_0.reshape(1, D)
    b1_1r, b2_1r = b1_1.reshape(1, D), b2_1.reshape(1, D)
    g1r, be1r = gamma_1.reshape(1, D), beta_1.reshape(1, D)
    batch2d = batch.reshape(1, N)

    agg0 = _get_sc_agg()(x, src, dst)
    z1 = _tc_layer(x, agg0[0], agg0[1], W1_0, b1_0r, W2_0, b2_0r, g0r, be0r)
    agg1 = _get_sc_agg()(z1, src, dst)
    z2, g1, g2 = _tc_final(z1, agg1[0], agg1[1], W1_1, b1_1r, W2_1, b2_1r,
                           g1r, be1r, batch2d)
    return (jnp.concatenate([z1, z2], axis=1), jnp.concatenate([g1, g2], axis=1))


# speedup vs baseline: 1.2280x; 1.2280x over previous
import functools

import jax
import jax.numpy as jnp
from jax import lax
from jax.experimental import pallas as pl
from jax.experimental.pallas import tpu as pltpu
from jax.experimental.pallas import tpu_sc as plsc

N = 10000
E = 320000
D = 128
G = 128

NC = 2
NS = 16
NW = NC * NS

CH = 128
NH = 2
HC = 40
NCHUNK = NH * HC
EPAD = NW * NCHUNK * CH
NPAD = 10240
RPW = NPAD // NS
ZB = RPW // 2
OPW = NPAD // NS


def _sc_agg_body(z_hbm, src_hbm, dst_hbm, out_hbm,
                 src_v, dst_v, rows_v, rows_w, agg_sh,
                 sem, sem2, ssem, ssem2):
    cid = lax.axis_index("c")
    sid = lax.axis_index("s")

    zeros16 = jnp.zeros((16,), jnp.float32)

    def zero_body(i, carry):
        for j in range(D // 16):
            rows_v[i, pl.ds(j * 16, 16)] = zeros16
        return carry

    lax.fori_loop(0, CH, zero_body, 0)
    for k in range(RPW // CH):
        pltpu.sync_copy(rows_v, agg_sh.at[pl.ds(sid * RPW + k * CH, CH)])

    plsc.subcore_barrier()

    def drain(buf, s):
        pltpu.make_async_copy(z_hbm.at[pl.ds(0, CH)], buf, s).wait()

    for h in range(NH):
        pltpu.sync_copy(src_hbm.at[cid, sid, h], src_v)
        pltpu.sync_copy(dst_hbm.at[cid, sid, h], dst_v)
        pltpu.async_copy(z_hbm.at[src_v.at[0]], rows_v, sem)
        pltpu.async_copy(z_hbm.at[src_v.at[1]], rows_w, sem2)

        def edge_body(k, carry):
            j = 2 * k
            drain(rows_v, sem)
            pltpu.async_copy(rows_v, agg_sh.at[dst_v.at[j]], ssem, add=True)
            drain(rows_w, sem2)
            pltpu.async_copy(rows_w, agg_sh.at[dst_v.at[j + 1]], ssem2,
                             add=True)
            drain(rows_v, ssem)
            pltpu.async_copy(z_hbm.at[src_v.at[(j + 2) % HC]], rows_v, sem)
            drain(rows_w, ssem2)
            pltpu.async_copy(z_hbm.at[src_v.at[(j + 3) % HC]], rows_w, sem2)
            return carry

        lax.fori_loop(0, HC // 2, edge_body, 0)
        drain(rows_v, sem)
        drain(rows_w, sem2)

    plsc.subcore_barrier()

    pltpu.sync_copy(agg_sh.at[pl.ds(sid * OPW, OPW)],
                    out_hbm.at[cid, pl.ds(sid * OPW, OPW)])


@functools.cache
def _get_sc_agg():
    mesh = plsc.VectorSubcoreMesh(core_axis_name="c", subcore_axis_name="s",
                                  num_cores=NC, num_subcores=NS)
    return pl.kernel(
        _sc_agg_body,
        out_type=jax.ShapeDtypeStruct((NC, NPAD, D), jnp.float32),
        mesh=mesh,
        scratch_types=[
            pltpu.VMEM((HC, CH), jnp.int32),
            pltpu.VMEM((HC, CH), jnp.int32),
            pltpu.VMEM((CH, D), jnp.float32),
            pltpu.VMEM((CH, D), jnp.float32),
            pltpu.VMEM_SHARED((NPAD, D), jnp.float32),
            pltpu.SemaphoreType.DMA,
            pltpu.SemaphoreType.DMA,
            pltpu.SemaphoreType.DMA,
            pltpu.SemaphoreType.DMA,
        ],
    )


def _tc_layer_body(z_ref, a0_ref, a1_ref, w1_ref, b1_ref, w2_ref, b2_ref,
                   gm_ref, bt_ref, out_ref):
    h = z_ref[...] + a0_ref[:N, :] + a1_ref[:N, :]
    h = jnp.maximum(jnp.dot(h, w1_ref[...],
                            preferred_element_type=jnp.float32) + b1_ref[...], 0.0)
    h = jnp.dot(h, w2_ref[...], preferred_element_type=jnp.float32) + b2_ref[...]
    h = jnp.maximum(h, 0.0)
    mean = jnp.mean(h, axis=0, keepdims=True)
    var = jnp.mean((h - mean) ** 2, axis=0, keepdims=True)
    out_ref[...] = (h - mean) * lax.rsqrt(var + 1e-5) * gm_ref[...] + bt_ref[...]


def _tc_layer(z, a0, a1, w1, b1, w2, b2, gm, bt):
    return pl.pallas_call(
        _tc_layer_body,
        out_shape=jax.ShapeDtypeStruct((N, D), jnp.float32),
    )(z, a0, a1, w1, b1, w2, b2, gm, bt)


def _tc_final_body(z1_ref, a0_ref, a1_ref, w1_ref, b1_ref, w2_ref, b2_ref,
                   gm_ref, bt_ref, batch_ref, z2_ref, g1_ref, g2_ref):
    h = z1_ref[...] + a0_ref[:N, :] + a1_ref[:N, :]
    h = jnp.maximum(jnp.dot(h, w1_ref[...],
                            preferred_element_type=jnp.float32) + b1_ref[...], 0.0)
    h = jnp.dot(h, w2_ref[...], preferred_element_type=jnp.float32) + b2_ref[...]
    h = jnp.maximum(h, 0.0)
    mean = jnp.mean(h, axis=0, keepdims=True)
    var = jnp.mean((h - mean) ** 2, axis=0, keepdims=True)
    z2 = (h - mean) * lax.rsqrt(var + 1e-5) * gm_ref[...] + bt_ref[...]
    z2_ref[...] = z2
    gid = lax.broadcasted_iota(jnp.int32, (G, 1), 0)
    onehot = (batch_ref[...] == gid).astype(jnp.float32)
    g1_ref[...] = jnp.dot(onehot, z1_ref[...], preferred_element_type=jnp.float32)
    g2_ref[...] = jnp.dot(onehot, z2, preferred_element_type=jnp.float32)


def _tc_final(z1, a0, a1, w1, b1, w2, b2, gm, bt, batch2d):
    return pl.pallas_call(
        _tc_final_body,
        out_shape=(
            jax.ShapeDtypeStruct((N, D), jnp.float32),
            jax.ShapeDtypeStruct((G, D), jnp.float32),
            jax.ShapeDtypeStruct((G, D), jnp.float32),
        ),
    )(z1, a0, a1, w1, b1, w2, b2, gm, bt, batch2d)


@jax.jit
def kernel(x, edge_index, batch, W1_0, b1_0, W2_0, b2_0, gamma_0, beta_0,
           W1_1, b1_1, W2_1, b2_1, gamma_1, beta_1):
    nchunks = NC * NS * NH * HC
    rpc = E // nchunks
    tpc = CH - rpc
    trash = N + (jnp.arange(nchunks * tpc, dtype=jnp.int32) % (NPAD - N))
    src = jnp.concatenate(
        [edge_index[0].reshape(nchunks, rpc),
         jnp.zeros((nchunks, tpc), jnp.int32)], axis=1)
    dst = jnp.concatenate(
        [edge_index[1].reshape(nchunks, rpc),
         trash.reshape(nchunks, tpc)], axis=1)
    src = src.reshape(NC, NS, NH, HC, CH)
    dst = dst.reshape(NC, NS, NH, HC, CH)

    b1_0r, b2_0r = b1_0.reshape(1, D), b2_0.reshape(1, D)
    g0r, be0r = gamma_0.reshape(1, D), beta_0.reshape(1, D)
    b1_1r, b2_1r = b1_1.reshape(1, D), b2_1.reshape(1, D)
    g1r, be1r = gamma_1.reshape(1, D), beta_1.reshape(1, D)
    batch2d = batch.reshape(1, N)

    agg0 = _get_sc_agg()(x, src, dst)
    z1 = _tc_layer(x, agg0[0], agg0[1], W1_0, b1_0r, W2_0, b2_0r, g0r, be0r)
    agg1 = _get_sc_agg()(z1, src, dst)
    z2, g1, g2 = _tc_final(z1, agg1[0], agg1[1], W1_1, b1_1r, W2_1, b2_1r,
                           g1r, be1r, batch2d)
    return (jnp.concatenate([z1, z2], axis=1), jnp.concatenate([g1, g2], axis=1))


# speedup vs baseline: 2.7711x; 2.2567x over previous
import functools

import jax
import jax.numpy as jnp
from jax import lax
from jax.experimental import pallas as pl
from jax.experimental.pallas import tpu as pltpu
from jax.experimental.pallas import tpu_sc as plsc

N = 10000
E = 320000
D = 128
G = 128

NC = 2
NS = 16
NW = NC * NS

CH = 128
NH = 2
HC = 40
NCHUNK = NH * HC
EPAD = NW * NCHUNK * CH
NPAD = 10240
RPW = NPAD // NS
ZB = RPW // 2
OPW = NPAD // NS


def _sc_agg_body(z_hbm, src_hbm, dst_hbm, out_hbm,
                 src_v, dst_v, rows_v, rows_w, agg_sh,
                 sem, sem2, ssem, ssem2):
    cid = lax.axis_index("c")
    sid = lax.axis_index("s")

    zeros16 = jnp.zeros((16,), jnp.float32)

    def zero_body(i, carry):
        for j in range(D // 16):
            rows_v[i, pl.ds(j * 16, 16)] = zeros16
        return carry

    lax.fori_loop(0, CH, zero_body, 0)
    for k in range(RPW // CH):
        pltpu.sync_copy(rows_v, agg_sh.at[pl.ds(sid * RPW + k * CH, CH)])

    plsc.subcore_barrier()

    def drain(buf, s):
        pltpu.make_async_copy(z_hbm.at[pl.ds(0, CH)], buf, s).wait()

    for h in range(NH):
        pltpu.sync_copy(src_hbm.at[cid, sid, h], src_v)
        pltpu.sync_copy(dst_hbm.at[cid, sid, h], dst_v)
        pltpu.async_copy(z_hbm.at[src_v.at[0]], rows_v, sem)
        pltpu.async_copy(z_hbm.at[src_v.at[1]], rows_w, sem2)

        def edge_body(k, carry):
            j = 2 * k
            drain(rows_v, sem)
            pltpu.async_copy(rows_v, agg_sh.at[dst_v.at[j]], ssem, add=True)
            drain(rows_w, sem2)
            pltpu.async_copy(rows_w, agg_sh.at[dst_v.at[j + 1]], ssem2,
                             add=True)
            drain(rows_v, ssem)
            pltpu.async_copy(z_hbm.at[src_v.at[(j + 2) % HC]], rows_v, sem)
            drain(rows_w, ssem2)
            pltpu.async_copy(z_hbm.at[src_v.at[(j + 3) % HC]], rows_w, sem2)
            return carry

        lax.fori_loop(0, HC // 2, edge_body, 0)
        drain(rows_v, sem)
        drain(rows_w, sem2)

    plsc.subcore_barrier()

    pltpu.sync_copy(agg_sh.at[pl.ds(sid * OPW, OPW)],
                    out_hbm.at[cid, pl.ds(sid * OPW, OPW)])


@functools.cache
def _get_sc_agg():
    mesh = plsc.VectorSubcoreMesh(core_axis_name="c", subcore_axis_name="s",
                                  num_cores=NC, num_subcores=NS)
    return pl.kernel(
        _sc_agg_body,
        out_type=jax.ShapeDtypeStruct((NC, NPAD, D), jnp.float32),
        mesh=mesh,
        scratch_types=[
            pltpu.VMEM((HC, CH), jnp.int32),
            pltpu.VMEM((HC, CH), jnp.int32),
            pltpu.VMEM((CH, D), jnp.float32),
            pltpu.VMEM((CH, D), jnp.float32),
            pltpu.VMEM_SHARED((NPAD, D), jnp.float32),
            pltpu.SemaphoreType.DMA,
            pltpu.SemaphoreType.DMA,
            pltpu.SemaphoreType.DMA,
            pltpu.SemaphoreType.DMA,
        ],
    )


def _tc_layer_body(z_ref, a0_ref, a1_ref, w1_ref, b1_ref, w2_ref, b2_ref,
                   gm_ref, bt_ref, out_ref):
    h = z_ref[...] + a0_ref[:N, :] + a1_ref[:N, :]
    h = jnp.maximum(jnp.dot(h, w1_ref[...],
                            preferred_element_type=jnp.float32) + b1_ref[...], 0.0)
    h = jnp.dot(h, w2_ref[...], preferred_element_type=jnp.float32) + b2_ref[...]
    h = jnp.maximum(h, 0.0)
    mean = jnp.mean(h, axis=0, keepdims=True)
    var = jnp.mean((h - mean) ** 2, axis=0, keepdims=True)
    out_ref[...] = (h - mean) * lax.rsqrt(var + 1e-5) * gm_ref[...] + bt_ref[...]


def _tc_layer(z, a0, a1, w1, b1, w2, b2, gm, bt):
    return pl.pallas_call(
        _tc_layer_body,
        out_shape=jax.ShapeDtypeStruct((N, D), jnp.float32),
    )(z, a0, a1, w1, b1, w2, b2, gm, bt)


def _tc_final_body(z1_ref, a0_ref, a1_ref, w1_ref, b1_ref, w2_ref, b2_ref,
                   gm_ref, bt_ref, batch_ref, z2_ref, g1_ref, g2_ref):
    h = z1_ref[...] + a0_ref[:N, :] + a1_ref[:N, :]
    h = jnp.maximum(jnp.dot(h, w1_ref[...],
                            preferred_element_type=jnp.float32) + b1_ref[...], 0.0)
    h = jnp.dot(h, w2_ref[...], preferred_element_type=jnp.float32) + b2_ref[...]
    h = jnp.maximum(h, 0.0)
    mean = jnp.mean(h, axis=0, keepdims=True)
    var = jnp.mean((h - mean) ** 2, axis=0, keepdims=True)
    z2 = (h - mean) * lax.rsqrt(var + 1e-5) * gm_ref[...] + bt_ref[...]
    z2_ref[...] = z2
    gid = lax.broadcasted_iota(jnp.int32, (G, 1), 0)
    onehot = (batch_ref[...] == gid).astype(jnp.float32)
    g1_ref[...] = jnp.dot(onehot, z1_ref[...], preferred_element_type=jnp.float32)
    g2_ref[...] = jnp.dot(onehot, z2, preferred_element_type=jnp.float32)


def _tc_final(z1, a0, a1, w1, b1, w2, b2, gm, bt, batch2d):
    return pl.pallas_call(
        _tc_final_body,
        out_shape=(
            jax.ShapeDtypeStruct((N, D), jnp.float32),
            jax.ShapeDtypeStruct((G, D), jnp.float32),
            jax.ShapeDtypeStruct((G, D), jnp.float32),
        ),
    )(z1, a0, a1, w1, b1, w2, b2, gm, bt, batch2d)


@jax.jit
def kernel(x, edge_index, batch, W1_0, b1_0, W2_0, b2_0, gamma_0, beta_0,
           W1_1, b1_1, W2_1, b2_1, gamma_1, beta_1):
    nchunks = NC * NS * NH * HC
    rpc = E // nchunks
    tpc = CH - rpc
    fill = jnp.arange(nchunks * tpc, dtype=jnp.int32)
    trash = N + fill % (NPAD - N)
    src = jnp.concatenate(
        [edge_index[0].reshape(nchunks, rpc),
         (fill * 131 % N).reshape(nchunks, tpc)], axis=1)
    dst = jnp.concatenate(
        [edge_index[1].reshape(nchunks, rpc),
         trash.reshape(nchunks, tpc)], axis=1)
    src = src.reshape(NC, NS, NH, HC, CH)
    dst = dst.reshape(NC, NS, NH, HC, CH)

    b1_0r, b2_0r = b1_0.reshape(1, D), b2_0.reshape(1, D)
    g0r, be0r = gamma_0.reshape(1, D), beta_0.reshape(1, D)
    b1_1r, b2_1r = b1_1.reshape(1, D), b2_1.reshape(1, D)
    g1r, be1r = gamma_1.reshape(1, D), beta_1.reshape(1, D)
    batch2d = batch.reshape(1, N)

    agg0 = _get_sc_agg()(x, src, dst)
    z1 = _tc_layer(x, agg0[0], agg0[1], W1_0, b1_0r, W2_0, b2_0r, g0r, be0r)
    agg1 = _get_sc_agg()(z1, src, dst)
    z2, g1, g2 = _tc_final(z1, agg1[0], agg1[1], W1_1, b1_1r, W2_1, b2_1r,
                           g1r, be1r, batch2d)
    return (jnp.concatenate([z1, z2], axis=1), jnp.concatenate([g1, g2], axis=1))
